# Initial kernel scaffold; baseline (speedup 1.0000x reference)
import jax
import jax.numpy as jnp
from jax.experimental import pallas as pl


def kernel(token_indices, embedding_matrix):
    raise NotImplementedError("write your pallas kernel here")


# speedup vs baseline: 2.9707x; 2.9707x over previous
import functools

import jax
import jax.numpy as jnp
from jax import lax
from jax.experimental import pallas as pl
from jax.experimental.pallas import tpu as pltpu
from jax.experimental.pallas import tpu_sc as plsc

BATCH = 4096
HIST = 50
D = 128
B_TOTAL = BATCH * HIST
NC, NS = 2, 16
NW = NC * NS
CHUNK = 128
B_PER_W = B_TOTAL // NW
N_CHUNKS = B_PER_W // CHUNK

_mesh = plsc.VectorSubcoreMesh(core_axis_name="c", subcore_axis_name="s")


@functools.partial(
    pl.kernel,
    out_type=jax.ShapeDtypeStruct((B_TOTAL, D), jnp.float32),
    mesh=_mesh,
    scratch_types=[
        pltpu.VMEM((N_CHUNKS, CHUNK), jnp.int32),
        pltpu.VMEM((CHUNK, D), jnp.float32),
        pltpu.SemaphoreType.DMA,
    ],
)
def _gather_kernel(table_hbm, idx_hbm, out_hbm, idx_v, rows_v, sem):
    wid = lax.axis_index("s") * NC + lax.axis_index("c")
    base = wid * B_PER_W
    pltpu.sync_copy(idx_hbm.at[wid], idx_v)

    @pl.loop(0, N_CHUNKS)
    def _chunk(j):
        pltpu.async_copy(table_hbm.at[idx_v.at[j]], rows_v, sem).wait()
        pltpu.sync_copy(rows_v, out_hbm.at[pl.ds(base + j * CHUNK, CHUNK)])


def kernel(token_indices, embedding_matrix):
    idx = token_indices.reshape(NW, N_CHUNKS, CHUNK).astype(jnp.int32)
    out = _gather_kernel(embedding_matrix, idx)
    return out.reshape(BATCH, HIST, D)


# speedup vs baseline: 3.2478x; 1.0933x over previous
import functools

import jax
import jax.numpy as jnp
from jax import lax
from jax.experimental import pallas as pl
from jax.experimental.pallas import tpu as pltpu
from jax.experimental.pallas import tpu_sc as plsc

BATCH = 4096
HIST = 50
D = 128
B_TOTAL = BATCH * HIST
NC, NS = 2, 16
NW = NC * NS
CHUNK = 128
B_PER_W = B_TOTAL // NW
N_CHUNKS = B_PER_W // CHUNK

_mesh = plsc.VectorSubcoreMesh(core_axis_name="c", subcore_axis_name="s")


@functools.partial(
    pl.kernel,
    out_type=jax.ShapeDtypeStruct((B_TOTAL, D), jnp.float32),
    mesh=_mesh,
    scratch_types=[
        pltpu.VMEM((N_CHUNKS, CHUNK), jnp.int32),
        pltpu.VMEM((2, CHUNK, D), jnp.float32),
        pltpu.SemaphoreType.DMA,
        pltpu.SemaphoreType.DMA,
        pltpu.SemaphoreType.DMA,
        pltpu.SemaphoreType.DMA,
    ],
)
def _gather_kernel(table_hbm, idx_hbm, out_hbm, idx_v, rows_v, g0, g1, o0, o1):
    wid = lax.axis_index("s") * NC + lax.axis_index("c")
    base = wid * B_PER_W
    pltpu.sync_copy(idx_hbm.at[wid], idx_v)
    gsem = (g0, g1)
    osem = (o0, o1)

    def gather(j, b):
        pltpu.async_copy(table_hbm.at[idx_v.at[j]], rows_v.at[b], gsem[b])

    def wait_gather(b):
        pltpu.make_async_copy(table_hbm.at[idx_v.at[0]], rows_v.at[b], gsem[b]).wait()

    def put(j, b):
        pltpu.async_copy(rows_v.at[b], out_hbm.at[pl.ds(base + j * CHUNK, CHUNK)], osem[b])

    def wait_put(b):
        pltpu.make_async_copy(rows_v.at[b], out_hbm.at[pl.ds(base, CHUNK)], osem[b]).wait()

    gather(0, 0)
    gather(1, 1)
    wait_gather(0)
    put(0, 0)
    wait_gather(1)
    put(1, 1)

    @pl.loop(1, N_CHUNKS // 2)
    def _pair(k):
        j = 2 * k
        wait_put(0)
        gather(j, 0)
        wait_put(1)
        gather(j + 1, 1)
        wait_gather(0)
        put(j, 0)
        wait_gather(1)
        put(j + 1, 1)

    wait_put(0)
    wait_put(1)


def kernel(token_indices, embedding_matrix):
    idx = token_indices.reshape(NW, N_CHUNKS, CHUNK).astype(jnp.int32)
    out = _gather_kernel(embedding_matrix, idx)
    return out.reshape(BATCH, HIST, D)


# speedup vs baseline: 5.7615x; 1.7739x over previous
import functools

import jax
import jax.numpy as jnp
from jax import lax
from jax.experimental import pallas as pl
from jax.experimental.pallas import tpu as pltpu
from jax.experimental.pallas import tpu_sc as plsc

BATCH = 4096
HIST = 50
D = 128
NC, NS = 2, 16
NW = NC * NS
B_PER_W = BATCH // NW
NBUF = 4
N_LOOP = B_PER_W // NBUF - 1

_mesh = plsc.VectorSubcoreMesh(core_axis_name="c", subcore_axis_name="s")


@functools.partial(
    pl.kernel,
    out_type=jax.ShapeDtypeStruct((BATCH, HIST, D), jnp.float32),
    mesh=_mesh,
    scratch_types=[
        pltpu.VMEM((B_PER_W, HIST), jnp.int32),
        pltpu.VMEM((NBUF, HIST, D), jnp.float32),
        [pltpu.SemaphoreType.DMA] * NBUF,
        [pltpu.SemaphoreType.DMA] * NBUF,
    ],
)
def _gather_kernel(table_hbm, idx_hbm, out_hbm, idx_v, rows_v, gsem, osem):
    wid = lax.axis_index("s") * NC + lax.axis_index("c")
    base = wid * B_PER_W
    pltpu.sync_copy(idx_hbm.at[pl.ds(base, B_PER_W)], idx_v)

    def gather(j, b):
        pltpu.async_copy(table_hbm.at[idx_v.at[j]], rows_v.at[b], gsem[b])

    def wait_gather(b):
        pltpu.make_async_copy(table_hbm.at[idx_v.at[0]], rows_v.at[b], gsem[b]).wait()

    def put(j, b):
        pltpu.async_copy(rows_v.at[b], out_hbm.at[base + j], osem[b])

    def wait_put(b):
        pltpu.make_async_copy(rows_v.at[b], out_hbm.at[base], osem[b]).wait()

    for b in range(NBUF):
        gather(b, b)
    for b in range(NBUF):
        wait_gather(b)
        put(b, b)

    @pl.loop(1, N_LOOP + 1)
    def _group(k):
        j0 = NBUF * k
        for b in range(NBUF):
            wait_put(b)
            gather(j0 + b, b)
        for b in range(NBUF):
            wait_gather(b)
            put(j0 + b, b)

    for b in range(NBUF):
        wait_put(b)


def kernel(token_indices, embedding_matrix):
    return _gather_kernel(embedding_matrix, token_indices.astype(jnp.int32))


# speedup vs baseline: 9.6002x; 1.6663x over previous
import functools

import jax
import jax.numpy as jnp
from jax import lax
from jax.experimental import pallas as pl
from jax.experimental.pallas import tpu as pltpu
from jax.experimental.pallas import tpu_sc as plsc

BATCH = 4096
HIST = 50
D = 128
NC, NS = 2, 16
NW = NC * NS
CHUNK = BATCH // NW

_mesh = plsc.VectorSubcoreMesh(core_axis_name="c", subcore_axis_name="s")


@functools.partial(
    pl.kernel,
    out_type=jax.ShapeDtypeStruct((HIST, BATCH, D), jnp.float32),
    mesh=_mesh,
    scratch_types=[
        pltpu.VMEM((HIST, CHUNK), jnp.int32),
        pltpu.VMEM((2, CHUNK, D), jnp.float32),
        [pltpu.SemaphoreType.DMA] * 2,
        [pltpu.SemaphoreType.DMA] * 2,
    ],
)
def _gather_kernel(table_hbm, idx_hbm, out_hbm, idx_v, rows_v, gsem, osem):
    wid = lax.axis_index("s") * NC + lax.axis_index("c")
    col = wid * CHUNK
    pltpu.sync_copy(idx_hbm.at[pl.ds(0, HIST), pl.ds(col, CHUNK)], idx_v)

    def gather(h, b):
        pltpu.async_copy(table_hbm.at[idx_v.at[h]], rows_v.at[b], gsem[b])

    def wait_gather(b):
        pltpu.make_async_copy(table_hbm.at[idx_v.at[0]], rows_v.at[b], gsem[b]).wait()

    def put(h, b):
        pltpu.async_copy(rows_v.at[b], out_hbm.at[h, pl.ds(col, CHUNK)], osem[b])

    def wait_put(b):
        pltpu.make_async_copy(rows_v.at[b], out_hbm.at[0, pl.ds(col, CHUNK)], osem[b]).wait()

    gather(0, 0)
    gather(1, 1)
    wait_gather(0)
    put(0, 0)
    wait_gather(1)
    put(1, 1)

    @pl.loop(1, HIST // 2)
    def _pair(k):
        h = 2 * k
        wait_put(0)
        gather(h, 0)
        wait_put(1)
        gather(h + 1, 1)
        wait_gather(0)
        put(h, 0)
        wait_gather(1)
        put(h + 1, 1)

    wait_put(0)
    wait_put(1)


def kernel(token_indices, embedding_matrix):
    idx_t = token_indices.T.astype(jnp.int32)
    out_t = _gather_kernel(embedding_matrix, idx_t)
    return out_t.transpose(1, 0, 2)


# speedup vs baseline: 10.4365x; 1.0871x over previous
import functools

import jax
import jax.numpy as jnp
from jax import lax
from jax.experimental import pallas as pl
from jax.experimental.pallas import tpu as pltpu
from jax.experimental.pallas import tpu_sc as plsc

BATCH = 4096
HIST = 50
D = 128
NC, NS = 2, 16
NW = NC * NS
CHUNK = BATCH // NW
NBUF = 5

_mesh = plsc.VectorSubcoreMesh(core_axis_name="c", subcore_axis_name="s")


@functools.partial(
    pl.kernel,
    out_type=jax.ShapeDtypeStruct((HIST, BATCH, D), jnp.float32),
    mesh=_mesh,
    scratch_types=[
        pltpu.VMEM((HIST, CHUNK), jnp.int32),
        pltpu.VMEM((NBUF, CHUNK, D), jnp.float32),
        [pltpu.SemaphoreType.DMA] * NBUF,
        [pltpu.SemaphoreType.DMA] * NBUF,
    ],
)
def _gather_kernel(table_hbm, idx_hbm, out_hbm, idx_v, rows_v, gsem, osem):
    wid = lax.axis_index("s") * NC + lax.axis_index("c")
    col = wid * CHUNK
    pltpu.sync_copy(idx_hbm.at[pl.ds(0, HIST), pl.ds(col, CHUNK)], idx_v)

    def gather(h, b):
        pltpu.async_copy(table_hbm.at[idx_v.at[h]], rows_v.at[b], gsem[b])

    def wait_gather(b):
        pltpu.make_async_copy(table_hbm.at[idx_v.at[0]], rows_v.at[b], gsem[b]).wait()

    def put(h, b):
        pltpu.async_copy(rows_v.at[b], out_hbm.at[h, pl.ds(col, CHUNK)], osem[b])

    def wait_put(b):
        pltpu.make_async_copy(rows_v.at[b], out_hbm.at[0, pl.ds(col, CHUNK)], osem[b]).wait()

    for b in range(NBUF):
        gather(b, b)
    for b in range(NBUF):
        wait_gather(b)
        put(b, b)

    @pl.loop(1, HIST // NBUF)
    def _group(k):
        h = NBUF * k
        for b in range(NBUF):
            wait_put(b)
            gather(h + b, b)
        for b in range(NBUF):
            wait_gather(b)
            put(h + b, b)

    for b in range(NBUF):
        wait_put(b)


def kernel(token_indices, embedding_matrix):
    idx_t = token_indices.T.astype(jnp.int32)
    out_t = _gather_kernel(embedding_matrix, idx_t)
    return out_t.transpose(1, 0, 2)


# speedup vs baseline: 10.6422x; 1.0197x over previous
import functools

import jax
import jax.numpy as jnp
from jax import lax
from jax.experimental import pallas as pl
from jax.experimental.pallas import tpu as pltpu
from jax.experimental.pallas import tpu_sc as plsc

BATCH = 4096
HIST = 50
D = 128
NC, NS = 2, 16
NW = NC * NS
COLS = BATCH // NW
CHUNK = 64
NBUF = 10

_mesh = plsc.VectorSubcoreMesh(core_axis_name="c", subcore_axis_name="s")


@functools.partial(
    pl.kernel,
    out_type=jax.ShapeDtypeStruct((HIST, BATCH, D), jnp.float32),
    mesh=_mesh,
    scratch_types=[
        pltpu.VMEM((HIST, COLS), jnp.int32),
        pltpu.VMEM((NBUF, CHUNK, D), jnp.float32),
        [pltpu.SemaphoreType.DMA] * NBUF,
        [pltpu.SemaphoreType.DMA] * NBUF,
    ],
)
def _gather_kernel(table_hbm, idx_hbm, out_hbm, idx_v, rows_v, gsem, osem):
    wid = lax.axis_index("s") * NC + lax.axis_index("c")
    col = wid * COLS
    pltpu.sync_copy(idx_hbm.at[pl.ds(0, HIST), pl.ds(col, COLS)], idx_v)

    def gather(h, half, b):
        pltpu.async_copy(
            table_hbm.at[idx_v.at[h, pl.ds(half * CHUNK, CHUNK)]],
            rows_v.at[b], gsem[b])

    def wait_gather(b):
        pltpu.make_async_copy(
            table_hbm.at[idx_v.at[0, pl.ds(0, CHUNK)]], rows_v.at[b], gsem[b]
        ).wait()

    def put(h, half, b):
        pltpu.async_copy(
            rows_v.at[b], out_hbm.at[h, pl.ds(col + half * CHUNK, CHUNK)],
            osem[b])

    def wait_put(b):
        pltpu.make_async_copy(
            rows_v.at[b], out_hbm.at[0, pl.ds(col, CHUNK)], osem[b]
        ).wait()

    for b in range(NBUF):
        gather(b // 2, b % 2, b)
    for b in range(NBUF):
        wait_gather(b)
        put(b // 2, b % 2, b)

    @pl.loop(1, 2 * HIST // NBUF)
    def _group(k):
        h0 = (NBUF // 2) * k
        for b in range(NBUF):
            wait_put(b)
            gather(h0 + b // 2, b % 2, b)
        for b in range(NBUF):
            wait_gather(b)
            put(h0 + b // 2, b % 2, b)

    for b in range(NBUF):
        wait_put(b)


def kernel(token_indices, embedding_matrix):
    idx_t = token_indices.T.astype(jnp.int32)
    out_t = _gather_kernel(embedding_matrix, idx_t)
    return out_t.transpose(1, 0, 2)
